# baseline (speedup 1.0000x reference)
import functools

import jax
import jax.numpy as jnp
from jax.experimental import pallas as pl
from jax.experimental.pallas import tpu as pltpu
from jax.experimental.pallas import tpu_sc as plsc

N_TOK = 9216
K_CB = 8192
D_DIM = 256
COMMIT_W = 0.25

NB = 256
GRID = N_TOK // NB


def _tc_body(z_ref, cb_ref, idx_ref, loss_ref, ppl_ref, counts_ref, acc_ref):
    i = pl.program_id(0)

    @pl.when(i == 0)
    def _init():
        counts_ref[...] = jnp.zeros_like(counts_ref)
        acc_ref[...] = jnp.zeros_like(acc_ref)

    z = z_ref[...]
    cb = cb_ref[...]
    s = jax.lax.dot_general(z, cb, (((1,), (1,)), ((), ())),
                            preferred_element_type=jnp.float32)
    z_sq = jnp.sum(z * z, axis=1, keepdims=True)
    c_sq = jnp.sum(cb * cb, axis=1)[None, :]
    d = (z_sq - 2.0 * s) + c_sq

    m = jnp.min(d, axis=1, keepdims=True)
    lanes = jax.lax.broadcasted_iota(jnp.int32, (NB, K_CB), 1)
    idx = jnp.min(jnp.where(d == m, lanes, K_CB), axis=1)
    idx_ref[...] = idx

    onehot = (lanes == idx[:, None]).astype(jnp.float32)
    counts_ref[...] += jnp.sum(onehot, axis=0, keepdims=True)
    acc_ref[...] += jnp.sum(m).reshape(1, 1)

    @pl.when(i == GRID - 1)
    def _fini():
        loss_ref[...] = acc_ref[...] * ((1.0 + COMMIT_W) / (N_TOK * D_DIM))
        p = counts_ref[...] * (1.0 / N_TOK)
        ent = -jnp.sum(p * jnp.log(p + 1e-10))
        ppl_ref[...] = jnp.exp(ent).reshape(1, 1)


def _tc_stage(z_e, codebook):
    return pl.pallas_call(
        _tc_body,
        grid=(GRID,),
        in_specs=[
            pl.BlockSpec((NB, D_DIM), lambda i: (i, 0)),
            pl.BlockSpec((K_CB, D_DIM), lambda i: (0, 0)),
        ],
        out_specs=[
            pl.BlockSpec((NB,), lambda i: (i,)),
            pl.BlockSpec((1, 1), lambda i: (0, 0)),
            pl.BlockSpec((1, 1), lambda i: (0, 0)),
        ],
        out_shape=[
            jax.ShapeDtypeStruct((N_TOK,), jnp.int32),
            jax.ShapeDtypeStruct((1, 1), jnp.float32),
            jax.ShapeDtypeStruct((1, 1), jnp.float32),
        ],
        scratch_shapes=[
            pltpu.VMEM((1, K_CB), jnp.float32),
            pltpu.VMEM((1, 1), jnp.float32),
        ],
        compiler_params=pltpu.CompilerParams(
            dimension_semantics=("arbitrary",),
        ),
    )(z_e, codebook)


def _sc_gather(codebook, idx):
    info = plsc.get_sparse_core_info()
    nw = info.num_cores * info.num_subcores
    bpw = N_TOK // nw
    mesh = plsc.VectorSubcoreMesh(core_axis_name="c", subcore_axis_name="s")

    @functools.partial(
        pl.kernel,
        mesh=mesh,
        out_type=jax.ShapeDtypeStruct((N_TOK, D_DIM), jnp.float32),
        scratch_types=[
            pltpu.VMEM((bpw,), jnp.int32),
            pltpu.VMEM((bpw, D_DIM), jnp.float32),
            pltpu.SemaphoreType.DMA,
        ],
    )
    def k(cb_hbm, idx_hbm, out_hbm, idx_v, rows_v, sem):
        wid = jax.lax.axis_index("s") * info.num_cores + jax.lax.axis_index("c")
        base = wid * bpw
        pltpu.sync_copy(idx_hbm.at[pl.ds(base, bpw)], idx_v)
        pltpu.async_copy(cb_hbm.at[idx_v], rows_v, sem).wait()
        pltpu.sync_copy(rows_v, out_hbm.at[pl.ds(base, bpw)])

    return k(codebook, idx)


def kernel(z_e, codebook):
    idx, loss, ppl = _tc_stage(z_e, codebook)
    z_q = _sc_gather(codebook, idx)
    return (z_q, loss.reshape(()), ppl.reshape(()), idx)


# speedup vs baseline: 1.5815x; 1.5815x over previous
import functools

import jax
import jax.numpy as jnp
from jax.experimental import pallas as pl
from jax.experimental.pallas import tpu as pltpu
from jax.experimental.pallas import tpu_sc as plsc

N_TOK = 9216
K_CB = 8192
D_DIM = 256
COMMIT_W = 0.25

NB = 256
GRID = N_TOK // NB


def _tc_body(z_ref, cb_ref, idx_ref, loss_ref, acc_ref, csq_ref, lanes_ref):
    i = pl.program_id(0)

    @pl.when(i == 0)
    def _init():
        acc_ref[...] = jnp.zeros_like(acc_ref)
        cb0 = cb_ref[...]
        csq_ref[...] = jnp.sum(cb0 * cb0, axis=1)[None, :]
        lanes_ref[...] = jax.lax.broadcasted_iota(
            jnp.int32, (1, K_CB), 1).astype(jnp.float32)

    z = z_ref[...]
    cb = cb_ref[...]
    s2 = jax.lax.dot_general(z + z, cb, (((1,), (1,)), ((), ())),
                             preferred_element_type=jnp.float32)
    z_sq = jnp.sum(z * z, axis=1, keepdims=True)
    d = (z_sq - s2) + csq_ref[...]

    m = jnp.min(d, axis=1, keepdims=True)
    idx_f = jnp.min(jnp.where(d == m, lanes_ref[...], float(K_CB)), axis=1)
    idx_ref[...] = idx_f.astype(jnp.int32)

    acc_ref[...] += jnp.sum(m).reshape(1, 1)

    @pl.when(i == GRID - 1)
    def _fini():
        loss_ref[...] = acc_ref[...] * ((1.0 + COMMIT_W) / (N_TOK * D_DIM))


def _tc_stage(z_e, codebook):
    return pl.pallas_call(
        _tc_body,
        grid=(GRID,),
        in_specs=[
            pl.BlockSpec((NB, D_DIM), lambda i: (i, 0)),
            pl.BlockSpec((K_CB, D_DIM), lambda i: (0, 0)),
        ],
        out_specs=[
            pl.BlockSpec((NB,), lambda i: (i,)),
            pl.BlockSpec((1, 1), lambda i: (0, 0)),
        ],
        out_shape=[
            jax.ShapeDtypeStruct((N_TOK,), jnp.int32),
            jax.ShapeDtypeStruct((1, 1), jnp.float32),
        ],
        scratch_shapes=[
            pltpu.VMEM((1, 1), jnp.float32),
            pltpu.VMEM((1, K_CB), jnp.float32),
            pltpu.VMEM((1, K_CB), jnp.float32),
        ],
        compiler_params=pltpu.CompilerParams(
            dimension_semantics=("arbitrary",),
        ),
    )(z_e, codebook)


def _sc_gather_hist(codebook, idx):
    info = plsc.get_sparse_core_info()
    nc, ns = info.num_cores, info.num_subcores
    nw = nc * ns
    bpw = N_TOK // nw
    kps = K_CB // ns
    mesh = plsc.VectorSubcoreMesh(core_axis_name="c", subcore_axis_name="s")

    @functools.partial(
        pl.kernel,
        mesh=mesh,
        out_type=[
            jax.ShapeDtypeStruct((N_TOK, D_DIM), jnp.float32),
            jax.ShapeDtypeStruct((nc, K_CB), jnp.float32),
        ],
        scratch_types=[
            pltpu.VMEM((bpw,), jnp.int32),
            pltpu.VMEM((bpw, D_DIM), jnp.float32),
            pltpu.VMEM((kps,), jnp.float32),
            pltpu.VMEM_SHARED((K_CB,), jnp.float32),
            pltpu.SemaphoreType.DMA,
        ],
    )
    def k(cb_hbm, idx_hbm, out_hbm, cnt_hbm, idx_v, rows_v, fill_v, hist_sh, sem):
        cid = jax.lax.axis_index("c")
        sid = jax.lax.axis_index("s")
        wid = sid * nc + cid
        base = wid * bpw
        pltpu.sync_copy(idx_hbm.at[pl.ds(base, bpw)], idx_v)
        for j in range(kps // 16):
            fill_v[pl.ds(j * 16, 16)] = jnp.zeros((16,), jnp.float32)
        pltpu.sync_copy(fill_v, hist_sh.at[pl.ds(sid * kps, kps)])
        gather = pltpu.async_copy(cb_hbm.at[idx_v], rows_v, sem)
        plsc.subcore_barrier()
        for j in range(bpw // 16):
            fill_v[pl.ds(j * 16, 16)] = jnp.ones((16,), jnp.float32)
        pltpu.sync_copy(fill_v.at[pl.ds(0, bpw)], hist_sh.at[idx_v], add=True)
        plsc.subcore_barrier()
        pltpu.sync_copy(hist_sh.at[pl.ds(sid * kps, kps)],
                        cnt_hbm.at[cid, pl.ds(sid * kps, kps)])
        gather.wait()
        pltpu.sync_copy(rows_v, out_hbm.at[pl.ds(base, bpw)])

    return k(codebook, idx)


def _ppl_body(cnt_ref, ppl_ref):
    counts = cnt_ref[0:1, :] + cnt_ref[1:2, :]
    p = counts * (1.0 / N_TOK)
    ent = -jnp.sum(p * jnp.log(p + 1e-10))
    ppl_ref[...] = jnp.exp(ent).reshape(1, 1)


def _ppl_stage(counts):
    return pl.pallas_call(
        _ppl_body,
        out_shape=jax.ShapeDtypeStruct((1, 1), jnp.float32),
    )(counts)


def kernel(z_e, codebook):
    idx, loss = _tc_stage(z_e, codebook)
    z_q, counts = _sc_gather_hist(codebook, idx)
    ppl = _ppl_stage(counts)
    return (z_q, loss.reshape(()), ppl.reshape(()), idx)
